# Initial kernel scaffold; baseline (speedup 1.0000x reference)
import jax
import jax.numpy as jnp
from jax.experimental import pallas as pl


def kernel(label_p, label_t, denselabel_p, denselabel_t):
    raise NotImplementedError("write your pallas kernel here")


# speedup vs baseline: 32.8480x; 32.8480x over previous
import jax
import jax.numpy as jnp
from jax import lax
from jax.experimental import pallas as pl
from jax.experimental.pallas import tpu as pltpu

_R, _C = 1024, 4096
_GRID = 8
_BR = _R // _GRID
_HI_BITS = 0x42C80000


def _ohem_body(lp_ref, lt_ref, dp_ref, dt_ref, out_ref, bits_ref, acc_ref):
    i = pl.program_id(0)

    dt = dt_ref[...]
    dp = dp_ref[...]
    pos = dt == 1.0
    neg = dt == 0.0
    q = jnp.where(pos, dp, 1.0 - dp)
    bce = -jnp.clip(jnp.log(q), -100.0, None)

    bits_ref[pl.ds(i * _BR, _BR), :] = jnp.where(
        neg, lax.bitcast_convert_type(bce, jnp.int32), jnp.int32(-1))

    @pl.when(i == 0)
    def _init():
        acc_ref[0] = 0.0
        acc_ref[1] = 0.0
        acc_ref[2] = 0.0

    acc_ref[0] += jnp.sum(jnp.where(pos, bce, 0.0))
    acc_ref[1] += jnp.sum(pos.astype(jnp.int32)).astype(jnp.float32)
    acc_ref[2] += jnp.sum(neg.astype(jnp.int32)).astype(jnp.float32)

    @pl.when(i == _GRID - 1)
    def _finalize():
        sum_pos = acc_ref[0]
        n_pos = acc_ref[1]
        n_neg = acc_ref[2]
        k = jnp.floor(0.7 * n_neg).astype(jnp.int32)

        def count_gt(t):
            def body(j, acc):
                blk = bits_ref[pl.ds(j * _BR, _BR), :]
                return acc + jnp.sum((blk > t).astype(jnp.int32))
            return lax.fori_loop(0, _GRID, body, jnp.int32(0))

        def step(_, lohi):
            lo, hi = lohi
            mid = lo + lax.div(hi - lo, 2)
            go_low = count_gt(mid) < k
            return (jnp.where(go_low, lo, mid + 1),
                    jnp.where(go_low, mid, hi))

        vbits, _ = lax.fori_loop(
            0, 31, step, (jnp.int32(0), jnp.int32(_HI_BITS)))

        def sums(j, carry):
            cnt, tot = carry
            blk = bits_ref[pl.ds(j * _BR, _BR), :]
            above = blk > vbits
            vals = lax.bitcast_convert_type(blk, jnp.float32)
            return (cnt + jnp.sum(above.astype(jnp.int32)),
                    tot + jnp.sum(jnp.where(above, vals, 0.0)))

        cnt_above, sum_above = lax.fori_loop(
            0, _GRID, sums, (jnp.int32(0), jnp.float32(0.0)))

        vk = lax.bitcast_convert_type(vbits, jnp.float32)
        kf = k.astype(jnp.float32)
        loss_neg = (sum_above + (kf - cnt_above.astype(jnp.float32)) * vk) / kf
        loss_pos = sum_pos / n_pos

        d = lp_ref[...] - lt_ref[...]
        mse = jnp.mean(d * d)
        out_ref[0, 0] = mse + loss_pos + loss_neg


def kernel(label_p, label_t, denselabel_p, denselabel_t):
    out = pl.pallas_call(
        _ohem_body,
        grid=(_GRID,),
        in_specs=[
            pl.BlockSpec((_R, 4), lambda i: (0, 0)),
            pl.BlockSpec((_R, 4), lambda i: (0, 0)),
            pl.BlockSpec((_BR, _C), lambda i: (i, 0)),
            pl.BlockSpec((_BR, _C), lambda i: (i, 0)),
        ],
        out_shape=jax.ShapeDtypeStruct((1, 1), jnp.float32),
        out_specs=pl.BlockSpec((1, 1), lambda i: (0, 0),
                               memory_space=pltpu.SMEM),
        scratch_shapes=[
            pltpu.VMEM((_R, _C), jnp.int32),
            pltpu.SMEM((4,), jnp.float32),
        ],
    )(label_p, label_t, denselabel_p, denselabel_t)
    return out[0, 0]
